# baseline (device time: 210832 ns/iter reference)
import jax
import jax.numpy as jnp
from jax import lax
from jax.experimental import pallas as pl
from jax.experimental.pallas import tpu as pltpu

N_DEV = 4
R_STATS = 128


def kernel(x, W):
    logits = jnp.dot(
        x.astype(jnp.bfloat16),
        W.astype(jnp.bfloat16),
        preferred_element_type=jnp.float32,
    ).astype(jnp.bfloat16)

    m_rows, n_per = logits.shape
    half = n_per // 2
    n_total = N_DEV * n_per

    my_pos = jnp.reshape(lax.axis_index("i").astype(jnp.int32), (1,))

    def make_transfer(p_ref, commX, send_sems, recv_sems, t, nbr):
        src = p_ref if t == 0 else commX.at[t - 1]
        return pltpu.make_async_remote_copy(
            src_ref=src,
            dst_ref=commX.at[t],
            send_sem=send_sems.at[t],
            recv_sem=recv_sems.at[t],
            device_id=(nbr,),
            device_id_type=pl.DeviceIdType.MESH,
        )

    def body(
        my_sref,
        logit_ref,
        out_ref,
        p_ref,
        commR,
        commL,
        stats_ref,
        sendR,
        recvR,
        sendL,
        recvL,
        st_send,
        st_recv,
    ):
        k = pl.program_id(0)
        my = my_sref[0]
        left = lax.rem(my + N_DEV - 1, N_DEV)
        right = lax.rem(my + 1, N_DEV)

        @pl.when(k == 0)
        def _step0():
            barrier = pltpu.get_barrier_semaphore()
            for nbr in (left, right):
                pl.semaphore_signal(
                    barrier,
                    inc=1,
                    device_id=(nbr,),
                    device_id_type=pl.DeviceIdType.MESH,
                )
            pl.semaphore_wait(barrier, 2)

            for r in range(0, m_rows, R_STATS):
                rows = pl.ds(r, R_STATS)
                blk = logit_ref[rows, :].astype(jnp.float32)
                m_r = jnp.max(blk, axis=1, keepdims=True)
                s_r = jnp.sum(jnp.exp(blk - m_r), axis=1, keepdims=True)
                stats_ref[0, rows, 0:1] = m_r
                stats_ref[0, rows, 1:2] = s_r

            for h in range(N_DEV - 1):
                rdma = pltpu.make_async_remote_copy(
                    src_ref=stats_ref.at[h],
                    dst_ref=stats_ref.at[h + 1],
                    send_sem=st_send.at[h],
                    recv_sem=st_recv.at[h],
                    device_id=(left,),
                    device_id_type=pl.DeviceIdType.MESH,
                )
                rdma.start()
                rdma.wait()

            M = stats_ref[0, :, 0:1]
            for d in range(1, N_DEV):
                M = jnp.maximum(M, stats_ref[d, :, 0:1])
            S = stats_ref[0, :, 1:2] * jnp.exp(stats_ref[0, :, 0:1] - M)
            for d in range(1, N_DEV):
                S = S + stats_ref[d, :, 1:2] * jnp.exp(stats_ref[d, :, 0:1] - M)
            inv_s = 1.0 / S

            for r in range(0, m_rows, R_STATS):
                rows = pl.ds(r, R_STATS)
                p_ref[rows, :] = (
                    jnp.exp(
                        logit_ref[rows, :].astype(jnp.float32) - M[r : r + R_STATS]
                    )
                    * inv_s[r : r + R_STATS]
                ).astype(jnp.bfloat16)

            make_transfer(
                p_ref.at[:, pl.ds(0, half)], commR, sendR, recvR, 0, right
            ).start()
            make_transfer(
                p_ref.at[:, pl.ds(half, half)], commL, sendL, recvL, 0, left
            ).start()

            out_ref[...] = p_ref[:, 0:half]

        @pl.when(k == 1)
        def _step1():
            out_ref[...] = p_ref[:, half:]

        for kk in range(2, 2 * N_DEV):
            h = kk // 2
            ring = kk % 2

            @pl.when(k == kk)
            def _step(h=h, ring=ring):
                commX = commR if ring == 0 else commL
                send_sems = sendR if ring == 0 else sendL
                recv_sems = recvR if ring == 0 else recvL
                nbr = right if ring == 0 else left
                psrc = (
                    p_ref.at[:, pl.ds(0, half)]
                    if ring == 0
                    else p_ref.at[:, pl.ds(half, half)]
                )
                make_transfer(psrc, commX, send_sems, recv_sems, h - 1, nbr).wait()
                if h <= N_DEV - 2:
                    make_transfer(
                        psrc, commX, send_sems, recv_sems, h, nbr
                    ).start()
                out_ref[...] = commX[h - 1]

    def out_map(k, my_sref):
        my = my_sref[0]
        h = k // 2
        ring = lax.rem(k, 2)
        origin = jnp.where(
            ring == 0,
            lax.rem(my - h + N_DEV, N_DEV),
            lax.rem(my + h, N_DEV),
        )
        return (0, origin * 2 + ring)

    grid_spec = pltpu.PrefetchScalarGridSpec(
        num_scalar_prefetch=1,
        grid=(2 * N_DEV,),
        in_specs=[pl.BlockSpec(memory_space=pltpu.MemorySpace.VMEM)],
        out_specs=pl.BlockSpec((m_rows, half), out_map),
        scratch_shapes=[
            pltpu.VMEM((m_rows, n_per), jnp.bfloat16),
            pltpu.VMEM((N_DEV - 1, m_rows, half), jnp.bfloat16),
            pltpu.VMEM((N_DEV - 1, m_rows, half), jnp.bfloat16),
            pltpu.VMEM((N_DEV, m_rows, 2), jnp.float32),
            pltpu.SemaphoreType.DMA((N_DEV - 1,)),
            pltpu.SemaphoreType.DMA((N_DEV - 1,)),
            pltpu.SemaphoreType.DMA((N_DEV - 1,)),
            pltpu.SemaphoreType.DMA((N_DEV - 1,)),
            pltpu.SemaphoreType.DMA((N_DEV - 1,)),
            pltpu.SemaphoreType.DMA((N_DEV - 1,)),
        ],
    )

    return pl.pallas_call(
        body,
        out_shape=jax.ShapeDtypeStruct((m_rows, n_total), jnp.bfloat16),
        grid_spec=grid_spec,
        compiler_params=pltpu.CompilerParams(
            collective_id=0,
            vmem_limit_bytes=60 * 1024 * 1024,
            dimension_semantics=("arbitrary",),
        ),
    )(my_pos, logits)


# device time: 207318 ns/iter; 1.0169x vs baseline; 1.0169x over previous
import jax
import jax.numpy as jnp
from jax import lax
from jax.experimental import pallas as pl
from jax.experimental.pallas import tpu as pltpu

N_DEV = 4
R_STATS = 128


def kernel(x, W):
    logits = jnp.dot(
        x.astype(jnp.bfloat16),
        W.astype(jnp.bfloat16),
        preferred_element_type=jnp.float32,
    ).astype(jnp.bfloat16)

    m_rows, n_per = logits.shape
    half = n_per // 2
    n_total = N_DEV * n_per

    def body(
        logit_ref,
        out_ref,
        p_ref,
        commR,
        commL,
        stats_ref,
        sendR,
        recvR,
        sendL,
        recvL,
        segR_send,
        segR_recv,
        segL_send,
        segL_recv,
        st_send,
        st_recv,
        copy_sems,
    ):
        my = lax.axis_index("i")
        left = lax.rem(my + N_DEV - 1, N_DEV)
        right = lax.rem(my + 1, N_DEV)

        barrier = pltpu.get_barrier_semaphore()
        for nbr in (left, right):
            pl.semaphore_signal(
                barrier,
                inc=1,
                device_id=(nbr,),
                device_id_type=pl.DeviceIdType.MESH,
            )
        pl.semaphore_wait(barrier, 2)

        for r in range(0, m_rows, R_STATS):
            rows = pl.ds(r, R_STATS)
            blk = logit_ref[rows, :].astype(jnp.float32)
            m_r = jnp.max(blk, axis=1, keepdims=True)
            s_r = jnp.sum(jnp.exp(blk - m_r), axis=1, keepdims=True)
            stats_ref[0, rows, 0:1] = m_r
            stats_ref[0, rows, 1:2] = s_r

        for h in range(N_DEV - 1):
            rdma = pltpu.make_async_remote_copy(
                src_ref=stats_ref.at[h],
                dst_ref=stats_ref.at[h + 1],
                send_sem=st_send.at[h],
                recv_sem=st_recv.at[h],
                device_id=(left,),
                device_id_type=pl.DeviceIdType.MESH,
            )
            rdma.start()
            rdma.wait()

        M = stats_ref[0, :, 0:1]
        for d in range(1, N_DEV):
            M = jnp.maximum(M, stats_ref[d, :, 0:1])
        S = stats_ref[0, :, 1:2] * jnp.exp(stats_ref[0, :, 0:1] - M)
        for d in range(1, N_DEV):
            S = S + stats_ref[d, :, 1:2] * jnp.exp(stats_ref[d, :, 0:1] - M)
        C = M + jnp.log(S)

        for r in range(0, m_rows, R_STATS):
            rows = pl.ds(r, R_STATS)
            p_ref[rows, :] = (
                jnp.exp(
                    logit_ref[rows, :].astype(jnp.float32) - C[r : r + R_STATS]
                )
            ).astype(jnp.bfloat16)
            s = r // R_STATS
            pltpu.make_async_remote_copy(
                src_ref=p_ref.at[rows, pl.ds(0, half)],
                dst_ref=commR.at[0, rows, :],
                send_sem=segR_send.at[s],
                recv_sem=segR_recv.at[s],
                device_id=(right,),
                device_id_type=pl.DeviceIdType.MESH,
            ).start()
            pltpu.make_async_remote_copy(
                src_ref=p_ref.at[rows, pl.ds(half, half)],
                dst_ref=commL.at[0, rows, :],
                send_sem=segL_send.at[s],
                recv_sem=segL_recv.at[s],
                device_id=(left,),
                device_id_type=pl.DeviceIdType.MESH,
            ).start()

        def store(src_ref, origin, which_half, width, sem):
            copy = pltpu.make_async_copy(
                src_ref,
                out_ref.at[:, pl.ds(origin * n_per + which_half * half, width)],
                sem,
            )
            copy.start()
            return copy

        own = store(p_ref, my, 0, n_per, copy_sems.at[0])
        own.wait()
        for s in range(m_rows // R_STATS):
            rows = pl.ds(s * R_STATS, R_STATS)
            pltpu.make_async_remote_copy(
                src_ref=p_ref.at[rows, pl.ds(0, half)],
                dst_ref=commR.at[0, rows, :],
                send_sem=segR_send.at[s],
                recv_sem=segR_recv.at[s],
                device_id=(right,),
                device_id_type=pl.DeviceIdType.MESH,
            ).wait()
            pltpu.make_async_remote_copy(
                src_ref=p_ref.at[rows, pl.ds(half, half)],
                dst_ref=commL.at[0, rows, :],
                send_sem=segL_send.at[s],
                recv_sem=segL_recv.at[s],
                device_id=(left,),
                device_id_type=pl.DeviceIdType.MESH,
            ).wait()

        for h in range(1, N_DEV - 1):
            rh = pltpu.make_async_remote_copy(
                src_ref=commR.at[h - 1],
                dst_ref=commR.at[h],
                send_sem=sendR.at[h],
                recv_sem=recvR.at[h],
                device_id=(right,),
                device_id_type=pl.DeviceIdType.MESH,
            )
            lh = pltpu.make_async_remote_copy(
                src_ref=commL.at[h - 1],
                dst_ref=commL.at[h],
                send_sem=sendL.at[h],
                recv_sem=recvL.at[h],
                device_id=(left,),
                device_id_type=pl.DeviceIdType.MESH,
            )
            rh.start()
            lh.start()
            cr = store(commR.at[h - 1], lax.rem(my - h + N_DEV, N_DEV), 0, half,
                       copy_sems.at[0])
            cl = store(commL.at[h - 1], lax.rem(my + h, N_DEV), 1, half,
                       copy_sems.at[1])
            cr.wait()
            cl.wait()
            rh.wait()
            lh.wait()

        cr = store(commR.at[N_DEV - 2], lax.rem(my - 3 + N_DEV, N_DEV), 0, half,
                   copy_sems.at[0])
        cl = store(commL.at[N_DEV - 2], lax.rem(my + 3, N_DEV), 1, half,
                   copy_sems.at[1])
        cr.wait()
        cl.wait()

    return pl.pallas_call(
        body,
        out_shape=jax.ShapeDtypeStruct((m_rows, n_total), jnp.bfloat16),
        in_specs=[pl.BlockSpec(memory_space=pltpu.VMEM)],
        out_specs=pl.BlockSpec(memory_space=pl.ANY),
        scratch_shapes=[
            pltpu.VMEM((m_rows, n_per), jnp.bfloat16),
            pltpu.VMEM((N_DEV - 1, m_rows, half), jnp.bfloat16),
            pltpu.VMEM((N_DEV - 1, m_rows, half), jnp.bfloat16),
            pltpu.VMEM((N_DEV, m_rows, 2), jnp.float32),
            pltpu.SemaphoreType.DMA((N_DEV - 1,)),
            pltpu.SemaphoreType.DMA((N_DEV - 1,)),
            pltpu.SemaphoreType.DMA((N_DEV - 1,)),
            pltpu.SemaphoreType.DMA((N_DEV - 1,)),
            pltpu.SemaphoreType.DMA((m_rows // R_STATS,)),
            pltpu.SemaphoreType.DMA((m_rows // R_STATS,)),
            pltpu.SemaphoreType.DMA((m_rows // R_STATS,)),
            pltpu.SemaphoreType.DMA((m_rows // R_STATS,)),
            pltpu.SemaphoreType.DMA((N_DEV - 1,)),
            pltpu.SemaphoreType.DMA((N_DEV - 1,)),
            pltpu.SemaphoreType.DMA((2,)),
        ],
        compiler_params=pltpu.CompilerParams(
            collective_id=0,
            vmem_limit_bytes=60 * 1024 * 1024,
        ),
    )(logits)


# device time: 204363 ns/iter; 1.0317x vs baseline; 1.0145x over previous
import jax
import jax.numpy as jnp
from jax import lax
from jax.experimental import pallas as pl
from jax.experimental.pallas import tpu as pltpu

N_DEV = 4
R_STATS = 128


def kernel(x, W):
    logits = jnp.dot(
        x.astype(jnp.bfloat16),
        W.astype(jnp.bfloat16),
        preferred_element_type=jnp.float32,
    ).astype(jnp.bfloat16)

    m_rows, n_per = logits.shape
    half = n_per // 2
    n_total = N_DEV * n_per

    def body(
        logit_ref,
        out_ref,
        p_ref,
        commR,
        commL,
        stats_ref,
        segR_send,
        segR_recv,
        segL_send,
        segL_recv,
        st_send,
        st_recv,
        copy_sems,
    ):
        n_seg = m_rows // R_STATS
        my = lax.axis_index("i")
        left = lax.rem(my + N_DEV - 1, N_DEV)
        right = lax.rem(my + 1, N_DEV)

        barrier = pltpu.get_barrier_semaphore()
        for nbr in (left, right):
            pl.semaphore_signal(
                barrier,
                inc=1,
                device_id=(nbr,),
                device_id_type=pl.DeviceIdType.MESH,
            )
        pl.semaphore_wait(barrier, 2)

        for r in range(0, m_rows, R_STATS):
            rows = pl.ds(r, R_STATS)
            blk = logit_ref[rows, :].astype(jnp.float32)
            m_r = jnp.max(blk, axis=1, keepdims=True)
            s_r = jnp.sum(jnp.exp(blk - m_r), axis=1, keepdims=True)
            stats_ref[0, rows, 0:1] = m_r
            stats_ref[0, rows, 1:2] = s_r

        for h in range(N_DEV - 1):
            rdma = pltpu.make_async_remote_copy(
                src_ref=stats_ref.at[h],
                dst_ref=stats_ref.at[h + 1],
                send_sem=st_send.at[h],
                recv_sem=st_recv.at[h],
                device_id=(left,),
                device_id_type=pl.DeviceIdType.MESH,
            )
            rdma.start()
            rdma.wait()

        M = stats_ref[0, :, 0:1]
        for d in range(1, N_DEV):
            M = jnp.maximum(M, stats_ref[d, :, 0:1])
        S = stats_ref[0, :, 1:2] * jnp.exp(stats_ref[0, :, 0:1] - M)
        for d in range(1, N_DEV):
            S = S + stats_ref[d, :, 1:2] * jnp.exp(stats_ref[d, :, 0:1] - M)
        C = M + jnp.log(S)

        def seg_rdma(ring, h, s):
            rows = pl.ds(s * R_STATS, R_STATS)
            commX = commR if ring == 0 else commL
            send_sems = segR_send if ring == 0 else segL_send
            recv_sems = segR_recv if ring == 0 else segL_recv
            if h == 0:
                src = p_ref.at[rows, pl.ds(0 if ring == 0 else half, half)]
            else:
                src = commX.at[h - 1, rows, :]
            return pltpu.make_async_remote_copy(
                src_ref=src,
                dst_ref=commX.at[h, rows, :],
                send_sem=send_sems.at[h * n_seg + s],
                recv_sem=recv_sems.at[h * n_seg + s],
                device_id=(right,) if ring == 0 else (left,),
                device_id_type=pl.DeviceIdType.MESH,
            )

        for s in range(n_seg):
            rows = pl.ds(s * R_STATS, R_STATS)
            p_ref[rows, :] = (
                jnp.exp(
                    logit_ref[rows, :].astype(jnp.float32)
                    - C[s * R_STATS : (s + 1) * R_STATS]
                )
            ).astype(jnp.bfloat16)
            seg_rdma(0, 0, s).start()
            seg_rdma(1, 0, s).start()

        def store(src_ref, origin, which_half, width, sem):
            copy = pltpu.make_async_copy(
                src_ref,
                out_ref.at[:, pl.ds(origin * n_per + which_half * half, width)],
                sem,
            )
            copy.start()
            return copy

        own = store(p_ref, my, 0, n_per, copy_sems.at[0])
        own.wait()

        for h in range(1, N_DEV - 1):
            for s in range(n_seg):
                seg_rdma(0, h - 1, s).wait()
                seg_rdma(0, h, s).start()
                seg_rdma(1, h - 1, s).wait()
                seg_rdma(1, h, s).start()
            cr = store(commR.at[h - 1], lax.rem(my - h + N_DEV, N_DEV), 0, half,
                       copy_sems.at[0])
            cl = store(commL.at[h - 1], lax.rem(my + h, N_DEV), 1, half,
                       copy_sems.at[1])
            cr.wait()
            cl.wait()

        for s in range(n_seg):
            seg_rdma(0, N_DEV - 2, s).wait()
            seg_rdma(1, N_DEV - 2, s).wait()
        cr = store(commR.at[N_DEV - 2], lax.rem(my - 3 + N_DEV, N_DEV), 0, half,
                   copy_sems.at[0])
        cl = store(commL.at[N_DEV - 2], lax.rem(my + 3, N_DEV), 1, half,
                   copy_sems.at[1])
        cr.wait()
        cl.wait()

    return pl.pallas_call(
        body,
        out_shape=jax.ShapeDtypeStruct((m_rows, n_total), jnp.bfloat16),
        in_specs=[pl.BlockSpec(memory_space=pltpu.VMEM)],
        out_specs=pl.BlockSpec(memory_space=pl.ANY),
        scratch_shapes=[
            pltpu.VMEM((m_rows, n_per), jnp.bfloat16),
            pltpu.VMEM((N_DEV - 1, m_rows, half), jnp.bfloat16),
            pltpu.VMEM((N_DEV - 1, m_rows, half), jnp.bfloat16),
            pltpu.VMEM((N_DEV, m_rows, 2), jnp.float32),
            pltpu.SemaphoreType.DMA(((N_DEV - 1) * (m_rows // R_STATS),)),
            pltpu.SemaphoreType.DMA(((N_DEV - 1) * (m_rows // R_STATS),)),
            pltpu.SemaphoreType.DMA(((N_DEV - 1) * (m_rows // R_STATS),)),
            pltpu.SemaphoreType.DMA(((N_DEV - 1) * (m_rows // R_STATS),)),
            pltpu.SemaphoreType.DMA((N_DEV - 1,)),
            pltpu.SemaphoreType.DMA((N_DEV - 1,)),
            pltpu.SemaphoreType.DMA((2,)),
        ],
        compiler_params=pltpu.CompilerParams(
            collective_id=0,
            vmem_limit_bytes=60 * 1024 * 1024,
        ),
    )(logits)


# device time: 198412 ns/iter; 1.0626x vs baseline; 1.0300x over previous
import jax
import jax.numpy as jnp
from jax import lax
from jax.experimental import pallas as pl
from jax.experimental.pallas import tpu as pltpu

N_DEV = 4
R_STATS = 128


def kernel(x, W):
    logits = jnp.dot(
        x.astype(jnp.bfloat16),
        W.astype(jnp.bfloat16),
        preferred_element_type=jnp.float32,
    ).astype(jnp.bfloat16)

    m_rows, n_per = logits.shape
    half = n_per // 2
    n_total = N_DEV * n_per

    def body(
        logit_ref,
        out_ref,
        p_ref,
        commR,
        commL,
        stats_ref,
        segR_send,
        segR_recv,
        segL_send,
        segL_recv,
        st_send,
        st_recv,
        copy_sems,
    ):
        n_seg = m_rows // R_STATS
        my = lax.axis_index("i")
        left = lax.rem(my + N_DEV - 1, N_DEV)
        right = lax.rem(my + 1, N_DEV)

        barrier = pltpu.get_barrier_semaphore()
        for nbr in (left, right):
            pl.semaphore_signal(
                barrier,
                inc=1,
                device_id=(nbr,),
                device_id_type=pl.DeviceIdType.MESH,
            )
        pl.semaphore_wait(barrier, 2)

        for r in range(0, m_rows, R_STATS):
            rows = pl.ds(r, R_STATS)
            blk = logit_ref[rows, :].astype(jnp.float32)
            m_r = jnp.max(blk, axis=1, keepdims=True)
            s_r = jnp.sum(jnp.exp(blk - m_r), axis=1, keepdims=True)
            stats_ref[0, rows, 0:1] = m_r
            stats_ref[0, rows, 1:2] = s_r

        a_left = pltpu.make_async_remote_copy(
            src_ref=stats_ref.at[0],
            dst_ref=stats_ref.at[1],
            send_sem=st_send.at[0],
            recv_sem=st_recv.at[0],
            device_id=(left,),
            device_id_type=pl.DeviceIdType.MESH,
        )
        a_right = pltpu.make_async_remote_copy(
            src_ref=stats_ref.at[0],
            dst_ref=stats_ref.at[2],
            send_sem=st_send.at[1],
            recv_sem=st_recv.at[1],
            device_id=(right,),
            device_id_type=pl.DeviceIdType.MESH,
        )
        a_left.start()
        a_right.start()
        a_left.wait()
        a_right.wait()
        b = pltpu.make_async_remote_copy(
            src_ref=stats_ref.at[1],
            dst_ref=stats_ref.at[3],
            send_sem=st_send.at[2],
            recv_sem=st_recv.at[2],
            device_id=(left,),
            device_id_type=pl.DeviceIdType.MESH,
        )
        b.start()
        b.wait()

        M = stats_ref[0, :, 0:1]
        for d in range(1, N_DEV):
            M = jnp.maximum(M, stats_ref[d, :, 0:1])
        S = stats_ref[0, :, 1:2] * jnp.exp(stats_ref[0, :, 0:1] - M)
        for d in range(1, N_DEV):
            S = S + stats_ref[d, :, 1:2] * jnp.exp(stats_ref[d, :, 0:1] - M)
        C = M + jnp.log(S)

        def seg_rdma(ring, h, s):
            rows = pl.ds(s * R_STATS, R_STATS)
            commX = commR if ring == 0 else commL
            send_sems = segR_send if ring == 0 else segL_send
            recv_sems = segR_recv if ring == 0 else segL_recv
            if h == 0:
                src = p_ref.at[rows, pl.ds(0 if ring == 0 else half, half)]
            else:
                src = commX.at[h - 1, rows, :]
            return pltpu.make_async_remote_copy(
                src_ref=src,
                dst_ref=commX.at[h, rows, :],
                send_sem=send_sems.at[h * n_seg + s],
                recv_sem=recv_sems.at[h * n_seg + s],
                device_id=(right,) if ring == 0 else (left,),
                device_id_type=pl.DeviceIdType.MESH,
            )

        for s in range(n_seg):
            rows = pl.ds(s * R_STATS, R_STATS)
            p_ref[rows, :] = (
                jnp.exp(
                    logit_ref[rows, :].astype(jnp.float32)
                    - C[s * R_STATS : (s + 1) * R_STATS]
                )
            ).astype(jnp.bfloat16)
            seg_rdma(0, 0, s).start()
            seg_rdma(1, 0, s).start()

        def store(src_ref, origin, which_half, width, sem):
            copy = pltpu.make_async_copy(
                src_ref,
                out_ref.at[:, pl.ds(origin * n_per + which_half * half, width)],
                sem,
            )
            copy.start()
            return copy

        own = store(p_ref, my, 0, n_per, copy_sems.at[0])
        own.wait()

        for h in range(1, N_DEV - 1):
            for s in range(n_seg):
                seg_rdma(0, h - 1, s).wait()
                seg_rdma(0, h, s).start()
                seg_rdma(1, h - 1, s).wait()
                seg_rdma(1, h, s).start()
            cr = store(commR.at[h - 1], lax.rem(my - h + N_DEV, N_DEV), 0, half,
                       copy_sems.at[0])
            cl = store(commL.at[h - 1], lax.rem(my + h, N_DEV), 1, half,
                       copy_sems.at[1])
            cr.wait()
            cl.wait()

        org_r = lax.rem(my - 3 + N_DEV, N_DEV)
        org_l = lax.rem(my + 3, N_DEV)
        for s in range(n_seg):
            rows = pl.ds(s * R_STATS, R_STATS)
            seg_rdma(0, N_DEV - 2, s).wait()
            cr = pltpu.make_async_copy(
                commR.at[N_DEV - 2, rows, :],
                out_ref.at[rows, pl.ds(org_r * n_per, half)],
                copy_sems.at[0],
            )
            cr.start()
            seg_rdma(1, N_DEV - 2, s).wait()
            cl = pltpu.make_async_copy(
                commL.at[N_DEV - 2, rows, :],
                out_ref.at[rows, pl.ds(org_l * n_per + half, half)],
                copy_sems.at[1],
            )
            cl.start()
            cr.wait()
            cl.wait()

    return pl.pallas_call(
        body,
        out_shape=jax.ShapeDtypeStruct((m_rows, n_total), jnp.bfloat16),
        in_specs=[pl.BlockSpec(memory_space=pltpu.VMEM)],
        out_specs=pl.BlockSpec(memory_space=pl.ANY),
        scratch_shapes=[
            pltpu.VMEM((m_rows, n_per), jnp.bfloat16),
            pltpu.VMEM((N_DEV - 1, m_rows, half), jnp.bfloat16),
            pltpu.VMEM((N_DEV - 1, m_rows, half), jnp.bfloat16),
            pltpu.VMEM((N_DEV, m_rows, 2), jnp.float32),
            pltpu.SemaphoreType.DMA(((N_DEV - 1) * (m_rows // R_STATS),)),
            pltpu.SemaphoreType.DMA(((N_DEV - 1) * (m_rows // R_STATS),)),
            pltpu.SemaphoreType.DMA(((N_DEV - 1) * (m_rows // R_STATS),)),
            pltpu.SemaphoreType.DMA(((N_DEV - 1) * (m_rows // R_STATS),)),
            pltpu.SemaphoreType.DMA((N_DEV - 1,)),
            pltpu.SemaphoreType.DMA((N_DEV - 1,)),
            pltpu.SemaphoreType.DMA((2,)),
        ],
        compiler_params=pltpu.CompilerParams(
            collective_id=0,
            vmem_limit_bytes=60 * 1024 * 1024,
        ),
    )(logits)
